# baseline (speedup 1.0000x reference)
import jax
import jax.numpy as jnp
from jax.experimental import pallas as pl
from jax.experimental.pallas import tpu as pltpu

_BN = 32768
_RB = 256


def _matvec_kernel(x_ref, w_ref, b_ref, out_ref, k_ref, acc):
    i = pl.program_id(0)
    wsum = jnp.sum(w_ref[...], axis=0, keepdims=True)
    bsum = jnp.sum(b_ref[...])
    col = jax.lax.dot_general(
        x_ref[...], wsum,
        dimension_numbers=(((1,), (1,)), ((), ())),
        preferred_element_type=jnp.float32)
    s = col.reshape(_BN // 128, 128) + bsum
    out_ref[...] = s

    @pl.when(i == 0)
    def _init():
        acc[0] = 0
    acc[0] = acc[0] + jnp.sum((s != 0.0).astype(jnp.float32)).astype(jnp.int32)
    k_ref[0, 0] = acc[0]


def _mask_kernel(s_ref, k_ref, o_ref, sm):
    j = pl.program_id(0)
    s = s_ref[...]
    nz = (s != 0.0)
    mi = nz.astype(jnp.float32)

    @pl.when(j == 0)
    def _init():
        sm[0] = 0

    k = k_ref[0, 0]
    start = jnp.maximum(k // 2, 1)
    d = jax.lax.broadcasted_iota(jnp.int32, (128, 128), 0)
    l = jax.lax.broadcasted_iota(jnp.int32, (128, 128), 1)
    tri = (d <= l).astype(jnp.float32)
    incl = jax.lax.dot(mi, tri,
                       preferred_element_type=jnp.float32)
    sel = (d == 127).astype(jnp.float32)
    rowcnt = jax.lax.dot(incl, sel,
                         preferred_element_type=jnp.float32)
    r2 = jax.lax.broadcasted_iota(jnp.int32, (_RB, _RB), 0)
    q2 = jax.lax.broadcasted_iota(jnp.int32, (_RB, _RB), 1)
    low = (q2 < r2).astype(jnp.float32)
    rowoff = jax.lax.dot(low, rowcnt,
                         preferred_element_type=jnp.float32)
    carry = sm[0].astype(jnp.float32)
    rank = carry + rowoff + (incl - mi)
    keep = nz & (rank >= start.astype(jnp.float32))
    o_ref[...] = jnp.where(keep, s, 0.0)
    sm[0] = sm[0] + jnp.sum(mi).astype(jnp.int32)


def kernel(x, W, b):
    N, D = x.shape
    R = N // 128
    b2d = b.reshape(1, b.shape[0])
    sums2d, kval = pl.pallas_call(
        _matvec_kernel,
        grid=(N // _BN,),
        in_specs=[
            pl.BlockSpec((_BN, D), lambda i: (i, 0)),
            pl.BlockSpec((W.shape[0], D), lambda i: (0, 0)),
            pl.BlockSpec((1, b.shape[0]), lambda i: (0, 0)),
        ],
        out_specs=[
            pl.BlockSpec((_BN // 128, 128), lambda i: (i, 0)),
            pl.BlockSpec(memory_space=pltpu.SMEM),
        ],
        out_shape=[
            jax.ShapeDtypeStruct((R, 128), jnp.float32),
            jax.ShapeDtypeStruct((1, 1), jnp.int32),
        ],
        scratch_shapes=[pltpu.SMEM((1,), jnp.int32)],
        compiler_params=pltpu.CompilerParams(
            dimension_semantics=("arbitrary",)),
    )(x, W, b2d)

    out2d = pl.pallas_call(
        _mask_kernel,
        grid=(R // _RB,),
        in_specs=[
            pl.BlockSpec((_RB, 128), lambda j: (j, 0)),
            pl.BlockSpec(memory_space=pltpu.SMEM),
        ],
        out_specs=pl.BlockSpec((_RB, 128), lambda j: (j, 0)),
        out_shape=jax.ShapeDtypeStruct((R, 128), jnp.float32),
        scratch_shapes=[pltpu.SMEM((1,), jnp.int32)],
        compiler_params=pltpu.CompilerParams(
            dimension_semantics=("arbitrary",)),
    )(sums2d, kval)
    return out2d.reshape(N)


# speedup vs baseline: 1.0781x; 1.0781x over previous
import jax
import jax.numpy as jnp
from jax.experimental import pallas as pl
from jax.experimental.pallas import tpu as pltpu

_BN = 32768
_RB = _BN // 128


def _fused_kernel(x_ref, w_ref, b_ref, o_ref, s_scr, r_scr, sm):
    p = pl.program_id(0)
    j = pl.program_id(1)

    @pl.when(p == 0)
    def _produce():
        @pl.when(j == 0)
        def _init():
            sm[0] = 0
        wsum = jnp.sum(w_ref[...], axis=0, keepdims=True)
        bsum = jnp.sum(b_ref[...])
        col = jax.lax.dot_general(
            x_ref[...], wsum,
            dimension_numbers=(((1,), (1,)), ((), ())),
            preferred_element_type=jnp.float32)
        s = col.reshape(_RB, 128) + bsum
        nz = (s != 0.0)
        mi = nz.astype(jnp.float32)
        d = jax.lax.broadcasted_iota(jnp.int32, (128, 128), 0)
        l = jax.lax.broadcasted_iota(jnp.int32, (128, 128), 1)
        tri = (d <= l).astype(jnp.float32)
        incl = jax.lax.dot(mi, tri,
                           preferred_element_type=jnp.float32)
        sel = (d == 127).astype(jnp.float32)
        rowcnt = jax.lax.dot(incl, sel,
                             preferred_element_type=jnp.float32)
        r2 = jax.lax.broadcasted_iota(jnp.int32, (_RB, _RB), 0)
        q2 = jax.lax.broadcasted_iota(jnp.int32, (_RB, _RB), 1)
        low = (q2 < r2).astype(jnp.float32)
        rowoff = jax.lax.dot(low, rowcnt,
                             preferred_element_type=jnp.float32)
        carry = sm[0].astype(jnp.float32)
        rank = carry + rowoff + (incl - mi)
        s_scr[pl.ds(j * _RB, _RB), :] = s
        r_scr[pl.ds(j * _RB, _RB), :] = rank
        sm[0] = sm[0] + jnp.sum(mi).astype(jnp.int32)

    @pl.when(p == 1)
    def _emit():
        k = sm[0]
        start = jnp.maximum(k // 2, 1).astype(jnp.float32)
        s = s_scr[pl.ds(j * _RB, _RB), :]
        rank = r_scr[pl.ds(j * _RB, _RB), :]
        keep = (s != 0.0) & (rank >= start)
        o_ref[...] = jnp.where(keep, s, 0.0)


def kernel(x, W, b):
    N, D = x.shape
    R = N // 128
    NB = N // _BN
    b2d = b.reshape(1, b.shape[0])
    out2d = pl.pallas_call(
        _fused_kernel,
        grid=(2, NB),
        in_specs=[
            pl.BlockSpec((_BN, D), lambda p, j: (j * (1 - p) + (NB - 1) * p, 0)),
            pl.BlockSpec((W.shape[0], D), lambda p, j: (0, 0)),
            pl.BlockSpec((1, b.shape[0]), lambda p, j: (0, 0)),
        ],
        out_specs=pl.BlockSpec((_RB, 128), lambda p, j: (j * p, 0)),
        out_shape=jax.ShapeDtypeStruct((R, 128), jnp.float32),
        scratch_shapes=[
            pltpu.VMEM((R, 128), jnp.float32),
            pltpu.VMEM((R, 128), jnp.float32),
            pltpu.SMEM((1,), jnp.int32),
        ],
        compiler_params=pltpu.CompilerParams(
            dimension_semantics=("arbitrary", "arbitrary")),
    )(x, W, b2d)
    return out2d.reshape(N)
